# Initial kernel scaffold; baseline (speedup 1.0000x reference)
import jax
import jax.numpy as jnp
from jax.experimental import pallas as pl


def kernel(x, edge_index, W0, b0, W1, b1):
    raise NotImplementedError("write your pallas kernel here")


# speedup vs baseline: 7.3786x; 7.3786x over previous
import functools

import jax
import jax.numpy as jnp
from jax import lax
from jax.experimental import pallas as pl
from jax.experimental.pallas import tpu as pltpu
from jax.experimental.pallas import tpu_sc as plsc

N_NODES = 10000
D_IN = 128
D_HID = 64
D_OUT = 128
N_EDGES = 320000

K = 80
NC = 2
NS = 16
NW = NC * NS
CHUNKS = N_EDGES // K
CPW = CHUNKS // NW
G = 5
CG = CPW // G

AGG_ROWS = 10240
ROWS_PER_TILE = AGG_ROWS // NS
ZROWS = 40

BLK = 1000


def _sc_agg(x, src3d, dst3d):
    mesh = plsc.VectorSubcoreMesh(core_axis_name="c", subcore_axis_name="s")

    @functools.partial(
        pl.kernel,
        mesh=mesh,
        out_type=jax.ShapeDtypeStruct((NC, AGG_ROWS, D_IN), jnp.float32),
        scratch_types=[
            pltpu.VMEM((CG, K), jnp.int32),
            pltpu.VMEM((CG, K), jnp.int32),
            pltpu.VMEM((K, D_IN), jnp.float32),
            pltpu.VMEM((ZROWS, D_IN), jnp.float32),
            pltpu.VMEM_SHARED((AGG_ROWS, D_IN), jnp.float32),
            pltpu.SemaphoreType.DMA,
        ],
    )
    def k(x_hbm, src_hbm, dst_hbm, out_hbm, sidx, didx, rows, zbuf, agg, sem):
        c = lax.axis_index("c")
        s = lax.axis_index("s")
        wid = s * NC + c

        zeros16 = jnp.zeros((16,), jnp.float32)

        def zrow(r, carry):
            def zcol(cc, carry2):
                zbuf[r, pl.ds(cc * 16, 16)] = zeros16
                return carry2
            return lax.fori_loop(0, D_IN // 16, zcol, carry)

        lax.fori_loop(0, ZROWS, zrow, 0)
        for i in range(ROWS_PER_TILE // ZROWS):
            pltpu.sync_copy(
                zbuf, agg.at[pl.ds(s * ROWS_PER_TILE + i * ZROWS, ZROWS)])
        plsc.subcore_barrier()

        def body(j, carry):
            pltpu.async_copy(x_hbm.at[sidx.at[j]], rows, sem).wait()
            pltpu.sync_copy(rows, agg.at[didx.at[j]], add=True)
            return carry

        for g in range(G):
            pltpu.sync_copy(src_hbm.at[wid, g], sidx)
            pltpu.sync_copy(dst_hbm.at[wid, g], didx)
            lax.fori_loop(0, CG, body, 0)
        plsc.subcore_barrier()

        pltpu.sync_copy(
            agg.at[pl.ds(s * ROWS_PER_TILE, ROWS_PER_TILE)],
            out_hbm.at[c, pl.ds(s * ROWS_PER_TILE, ROWS_PER_TILE)])

    return k(x, src3d, dst3d)


def _mlp_body(x_ref, p0_ref, p1_ref, w0t_ref, b0_ref, w1t_ref, b1_ref, o_ref):
    h = x_ref[...] + p0_ref[...] + p1_ref[...]
    h = jnp.dot(h, w0t_ref[...], preferred_element_type=jnp.float32)
    h = jnp.maximum(h + b0_ref[...], 0.0)
    o_ref[...] = (
        jnp.dot(h, w1t_ref[...], preferred_element_type=jnp.float32)
        + b1_ref[...])


def _mlp(x, p0, p1, w0t, b0, w1t, b1):
    return pl.pallas_call(
        _mlp_body,
        grid=(N_NODES // BLK,),
        in_specs=[
            pl.BlockSpec((BLK, D_IN), lambda i: (i, 0)),
            pl.BlockSpec((BLK, D_IN), lambda i: (i, 0)),
            pl.BlockSpec((BLK, D_IN), lambda i: (i, 0)),
            pl.BlockSpec((D_IN, D_HID), lambda i: (0, 0)),
            pl.BlockSpec((1, D_HID), lambda i: (0, 0)),
            pl.BlockSpec((D_HID, D_OUT), lambda i: (0, 0)),
            pl.BlockSpec((1, D_OUT), lambda i: (0, 0)),
        ],
        out_specs=pl.BlockSpec((BLK, D_OUT), lambda i: (i, 0)),
        out_shape=jax.ShapeDtypeStruct((N_NODES, D_OUT), jnp.float32),
    )(x, p0, p1, w0t, b0, w1t, b1)


def kernel(x, edge_index, W0, b0, W1, b1):
    src = edge_index[0].astype(jnp.int32)
    dst = edge_index[1].astype(jnp.int32)
    src3d = src.reshape(NW, G, CG, K)
    dst3d = dst.reshape(NW, G, CG, K)
    parts = _sc_agg(x, src3d, dst3d)
    p0 = parts[0, :N_NODES]
    p1 = parts[1, :N_NODES]
    return _mlp(x, p0, p1,
                W0.T, b0.reshape(1, D_HID), W1.T, b1.reshape(1, D_OUT))


# speedup vs baseline: 10.8213x; 1.4666x over previous
import functools

import jax
import jax.numpy as jnp
from jax import lax
from jax.experimental import pallas as pl
from jax.experimental.pallas import tpu as pltpu
from jax.experimental.pallas import tpu_sc as plsc

N_NODES = 10000
D_IN = 128
D_HID = 64
D_OUT = 128
N_EDGES = 320000

K = 80
NC = 2
NS = 16
NW = NC * NS
CHUNKS = N_EDGES // K
CPW = CHUNKS // NW
G = 5
CG = CPW // G

AGG_ROWS = 10240
ROWS_PER_TILE = AGG_ROWS // NS
NBUF = 2

BLK = 1000


def _sc_agg(x, src3d, dst3d):
    mesh = plsc.VectorSubcoreMesh(core_axis_name="c", subcore_axis_name="s")

    @functools.partial(
        pl.kernel,
        mesh=mesh,
        out_type=jax.ShapeDtypeStruct((NC, AGG_ROWS, D_IN), jnp.float32),
        scratch_types=[
            pltpu.VMEM((CG, K), jnp.int32),
            pltpu.VMEM((CG, K), jnp.int32),
            pltpu.VMEM((NBUF, K, D_IN), jnp.float32),
            pltpu.VMEM_SHARED((AGG_ROWS, D_IN), jnp.float32),
            pltpu.SemaphoreType.DMA,
            pltpu.SemaphoreType.DMA,
            pltpu.SemaphoreType.DMA,
        ],
    )
    def k(x_hbm, src_hbm, dst_hbm, out_hbm, sidx, didx, rows, agg,
          gsem0, gsem1, ssem):
        c = lax.axis_index("c")
        s = lax.axis_index("s")
        wid = s * NC + c
        gsems = (gsem0, gsem1)

        zeros16 = jnp.zeros((16,), jnp.float32)

        def zrow(r, carry):
            def zcol(cc, carry2):
                rows[0, r, pl.ds(cc * 16, 16)] = zeros16
                return carry2
            return lax.fori_loop(0, D_IN // 16, zcol, carry)

        lax.fori_loop(0, K, zrow, 0)
        for i in range(ROWS_PER_TILE // K):
            pltpu.sync_copy(
                rows.at[0], agg.at[pl.ds(s * ROWS_PER_TILE + i * K, K)])
        plsc.subcore_barrier()

        def start_gather(j, b):
            pltpu.async_copy(x_hbm.at[sidx.at[j]], rows.at[b], gsems[b])

        def wait_gather(j, b):
            pltpu.make_async_copy(
                x_hbm.at[sidx.at[j]], rows.at[b], gsems[b]).wait()

        def scatter(j, b):
            pltpu.async_copy(rows.at[b], agg.at[didx.at[j]], ssem,
                             add=True).wait()

        for g in range(G):
            pltpu.sync_copy(src_hbm.at[wid, g], sidx)
            pltpu.sync_copy(dst_hbm.at[wid, g], didx)
            for b in range(NBUF):
                start_gather(b, b)

            def body(t, carry):
                j0 = t * NBUF
                for b in range(NBUF):
                    wait_gather(j0 + b, b)
                    scatter(j0 + b, b)
                    start_gather(j0 + b + NBUF, b)
                return carry

            steady = (CG - NBUF) // NBUF
            lax.fori_loop(0, steady, body, 0)
            for j in range(steady * NBUF, CG):
                b = j % NBUF
                wait_gather(j, b)
                scatter(j, b)
                nxt = j + NBUF
                if nxt < CG:
                    start_gather(nxt, b)
        plsc.subcore_barrier()

        pltpu.sync_copy(
            agg.at[pl.ds(s * ROWS_PER_TILE, ROWS_PER_TILE)],
            out_hbm.at[c, pl.ds(s * ROWS_PER_TILE, ROWS_PER_TILE)])

    return k(x, src3d, dst3d)


def _mlp_body(x_ref, p0_ref, p1_ref, w0t_ref, b0_ref, w1t_ref, b1_ref, o_ref):
    h = x_ref[...] + p0_ref[...] + p1_ref[...]
    h = jnp.dot(h, w0t_ref[...], preferred_element_type=jnp.float32)
    h = jnp.maximum(h + b0_ref[...], 0.0)
    o_ref[...] = (
        jnp.dot(h, w1t_ref[...], preferred_element_type=jnp.float32)
        + b1_ref[...])


def _mlp(x, p0, p1, w0t, b0, w1t, b1):
    return pl.pallas_call(
        _mlp_body,
        grid=(N_NODES // BLK,),
        in_specs=[
            pl.BlockSpec((BLK, D_IN), lambda i: (i, 0)),
            pl.BlockSpec((BLK, D_IN), lambda i: (i, 0)),
            pl.BlockSpec((BLK, D_IN), lambda i: (i, 0)),
            pl.BlockSpec((D_IN, D_HID), lambda i: (0, 0)),
            pl.BlockSpec((1, D_HID), lambda i: (0, 0)),
            pl.BlockSpec((D_HID, D_OUT), lambda i: (0, 0)),
            pl.BlockSpec((1, D_OUT), lambda i: (0, 0)),
        ],
        out_specs=pl.BlockSpec((BLK, D_OUT), lambda i: (i, 0)),
        out_shape=jax.ShapeDtypeStruct((N_NODES, D_OUT), jnp.float32),
    )(x, p0, p1, w0t, b0, w1t, b1)


def kernel(x, edge_index, W0, b0, W1, b1):
    src = edge_index[0].astype(jnp.int32)
    dst = edge_index[1].astype(jnp.int32)
    src3d = src.reshape(NW, G, CG, K)
    dst3d = dst.reshape(NW, G, CG, K)
    parts = _sc_agg(x, src3d, dst3d)
    p0 = parts[0, :N_NODES]
    p1 = parts[1, :N_NODES]
    return _mlp(x, p0, p1,
                W0.T, b0.reshape(1, D_HID), W1.T, b1.reshape(1, D_OUT))


# speedup vs baseline: 11.2954x; 1.0438x over previous
import functools

import jax
import jax.numpy as jnp
from jax import lax
from jax.experimental import pallas as pl
from jax.experimental.pallas import tpu as pltpu
from jax.experimental.pallas import tpu_sc as plsc

N_NODES = 10000
D_IN = 128
D_HID = 64
D_OUT = 128
N_EDGES = 320000

K = 80
NC = 2
NS = 16
NW = NC * NS
CHUNKS = N_EDGES // K
CPW = CHUNKS // NW
G = 5
CG = CPW // G

AGG_ROWS = 10240
ROWS_PER_TILE = AGG_ROWS // NS
NBUF = 2

BLK = 1000


def _sc_agg(x, src3d, dst3d):
    mesh = plsc.VectorSubcoreMesh(core_axis_name="c", subcore_axis_name="s")

    @functools.partial(
        pl.kernel,
        mesh=mesh,
        out_type=jax.ShapeDtypeStruct((NC, AGG_ROWS, D_IN), jnp.float32),
        scratch_types=[
            pltpu.VMEM((CG, K), jnp.int32),
            pltpu.VMEM((CG, K), jnp.int32),
            pltpu.VMEM((NBUF, K, D_IN), jnp.float32),
            pltpu.VMEM_SHARED((AGG_ROWS, D_IN), jnp.float32),
            pltpu.SemaphoreType.DMA,
            pltpu.SemaphoreType.DMA,
            pltpu.SemaphoreType.DMA,
        ],
    )
    def k(x_hbm, src_hbm, dst_hbm, out_hbm, sidx, didx, rows, agg,
          gsem0, gsem1, ssem):
        c = lax.axis_index("c")
        s = lax.axis_index("s")
        wid = s * NC + c
        gsems = (gsem0, gsem1)

        zeros16 = jnp.zeros((16,), jnp.float32)

        def zrow(r, carry):
            def zcol(cc, carry2):
                rows[0, r, pl.ds(cc * 16, 16)] = zeros16
                return carry2
            return lax.fori_loop(0, D_IN // 16, zcol, carry)

        lax.fori_loop(0, K, zrow, 0)
        for i in range(ROWS_PER_TILE // K):
            pltpu.sync_copy(
                rows.at[0], agg.at[pl.ds(s * ROWS_PER_TILE + i * K, K)])
        plsc.subcore_barrier()

        def start_gather(j, b):
            pltpu.async_copy(x_hbm.at[sidx.at[j]], rows.at[b], gsems[b])

        def wait_gather(j, b):
            pltpu.make_async_copy(
                x_hbm.at[sidx.at[j]], rows.at[b], gsems[b]).wait()

        def scatter(j, b):
            pltpu.async_copy(rows.at[b], agg.at[didx.at[j]], ssem,
                             add=True).wait()

        for g in range(G):
            pltpu.sync_copy(src_hbm.at[wid, g], sidx)
            pltpu.sync_copy(dst_hbm.at[wid, g], didx)
            for b in range(NBUF):
                start_gather(b, b)

            def body(t, carry):
                j0 = t * NBUF
                for b in range(NBUF):
                    wait_gather(j0 + b, b)
                    scatter(j0 + b, b)
                    start_gather(j0 + b + NBUF, b)
                return carry

            steady = (CG - NBUF) // NBUF
            lax.fori_loop(0, steady, body, 0)
            for j in range(steady * NBUF, CG):
                b = j % NBUF
                wait_gather(j, b)
                scatter(j, b)
                nxt = j + NBUF
                if nxt < CG:
                    start_gather(nxt, b)
        plsc.subcore_barrier()

        pltpu.sync_copy(
            agg.at[pl.ds(s * ROWS_PER_TILE, ROWS_PER_TILE)],
            out_hbm.at[c, pl.ds(s * ROWS_PER_TILE, ROWS_PER_TILE)])

    return k(x, src3d, dst3d)


def _mlp_body(x_ref, p0_ref, p1_ref, w0t_ref, b0_ref, w1t_ref, b1_ref, o_ref):
    h = x_ref[...] + p0_ref[0] + p1_ref[0]
    h = jnp.dot(h, w0t_ref[...], preferred_element_type=jnp.float32)
    h = jnp.maximum(h + b0_ref[...], 0.0)
    o_ref[...] = (
        jnp.dot(h, w1t_ref[...], preferred_element_type=jnp.float32)
        + b1_ref[...])


def _mlp(x, parts, w0t, b0, w1t, b1):
    return pl.pallas_call(
        _mlp_body,
        grid=(N_NODES // BLK,),
        in_specs=[
            pl.BlockSpec((BLK, D_IN), lambda i: (i, 0)),
            pl.BlockSpec((1, BLK, D_IN), lambda i: (0, i, 0)),
            pl.BlockSpec((1, BLK, D_IN), lambda i: (1, i, 0)),
            pl.BlockSpec((D_IN, D_HID), lambda i: (0, 0)),
            pl.BlockSpec((1, D_HID), lambda i: (0, 0)),
            pl.BlockSpec((D_HID, D_OUT), lambda i: (0, 0)),
            pl.BlockSpec((1, D_OUT), lambda i: (0, 0)),
        ],
        out_specs=pl.BlockSpec((BLK, D_OUT), lambda i: (i, 0)),
        out_shape=jax.ShapeDtypeStruct((N_NODES, D_OUT), jnp.float32),
    )(x, parts, parts, w0t, b0, w1t, b1)


def kernel(x, edge_index, W0, b0, W1, b1):
    src = edge_index[0].astype(jnp.int32)
    dst = edge_index[1].astype(jnp.int32)
    src3d = src.reshape(NW, G, CG, K)
    dst3d = dst.reshape(NW, G, CG, K)
    parts = _sc_agg(x, src3d, dst3d)
    return _mlp(x, parts,
                W0.T, b0.reshape(1, D_HID), W1.T, b1.reshape(1, D_OUT))
